# Initial kernel scaffold; baseline (speedup 1.0000x reference)
import jax
import jax.numpy as jnp
from jax.experimental import pallas as pl


def kernel(scores):
    raise NotImplementedError("write your pallas kernel here")


# speedup vs baseline: 14.8591x; 14.8591x over previous
import functools

import jax
import jax.numpy as jnp
from jax.experimental import pallas as pl
from jax.experimental.pallas import tpu as pltpu

_K = 256
_ROWS = 8


def _sortable_i32(x):
    b = jax.lax.bitcast_convert_type(x, jnp.int32)
    return b ^ ((b >> 31) & jnp.int32(0x7FFFFFFF))


def _tc_body(s_ref, p_ref, w_ref):
    s = s_ref[...]
    rows, n = s.shape

    m = jnp.max(s, axis=1, keepdims=True)
    e = jnp.exp(s - m)
    z = jnp.sum(e, axis=1, keepdims=True)
    p_ref[...] = e / z

    l = jnp.sign(s) * (1.0 - s)

    kw = _sortable_i32(s)
    kl = _sortable_i32(l)

    int_min = jnp.int32(-(2 ** 31))
    int_max = jnp.int32(2 ** 31 - 1)
    lo0 = jnp.full((rows, 1), int_min, jnp.int32)
    hi0 = jnp.full((rows, 1), int_max, jnp.int32)

    kk = jnp.int32(_K)

    def vstep(_, carry):
        lw, hw, ll, hl = carry
        mw = (lw >> 1) + (hw >> 1) + (lw & hw & 1)
        ml = (ll >> 1) + (hl >> 1) + (ll & hl & 1)
        cw = jnp.sum((kw >= mw).astype(jnp.int32), axis=1, keepdims=True)
        cl = jnp.sum((kl >= ml).astype(jnp.int32), axis=1, keepdims=True)
        pw = cw >= kk
        pl_ = cl >= kk
        lw = jnp.where(pw, mw, lw)
        hw = jnp.where(pw, hw, mw)
        ll = jnp.where(pl_, ml, ll)
        hl = jnp.where(pl_, hl, ml)
        return lw, hw, ll, hl

    lw, _, ll, _ = jax.lax.fori_loop(0, 32, vstep, (lo0, hi0, lo0, hi0))

    eqw = kw == lw
    eql = kl == ll
    cgtw = jnp.sum((kw > lw).astype(jnp.int32), axis=1, keepdims=True)
    cgtl = jnp.sum((kl > ll).astype(jnp.int32), axis=1, keepdims=True)
    rw = kk - cgtw
    rl = kk - cgtl

    idx = jax.lax.broadcasted_iota(jnp.int32, (rows, n), 1)

    li0 = jnp.full((rows, 1), -1, jnp.int32)
    hi0i = jnp.full((rows, 1), n - 1, jnp.int32)

    def istep(_, carry):
        liw, hiw, lil, hil = carry
        miw = (liw + hiw) >> 1
        mil = (lil + hil) >> 1
        cw = jnp.sum((eqw & (idx <= miw)).astype(jnp.int32), axis=1,
                     keepdims=True)
        cl = jnp.sum((eql & (idx <= mil)).astype(jnp.int32), axis=1,
                     keepdims=True)
        pw = cw >= rw
        pl_ = cl >= rl
        hiw = jnp.where(pw, miw, hiw)
        liw = jnp.where(pw, liw, miw)
        hil = jnp.where(pl_, mil, hil)
        lil = jnp.where(pl_, lil, mil)
        return liw, hiw, lil, hil

    _, itw, _, itl = jax.lax.fori_loop(0, 15, istep, (li0, hi0i, li0, hi0i))

    maskw = (kw > lw) | (eqw & (idx <= itw))
    maskl = (kl > ll) | (eql & (idx <= itl))

    ew = jnp.where(maskw, e, 0.0)
    zw = jnp.sum(ew, axis=1, keepdims=True)
    w_ref[:, 0:n] = ew / zw

    ml_ = jnp.max(l, axis=1, keepdims=True)
    el = jnp.where(maskl, jnp.exp(l - ml_), 0.0)
    zl = jnp.sum(el, axis=1, keepdims=True)
    w_ref[:, n:2 * n] = el / zl


@jax.jit
def kernel(scores):
    b, n = scores.shape
    grid = b // _ROWS
    p_out, w_out = pl.pallas_call(
        _tc_body,
        grid=(grid,),
        in_specs=[pl.BlockSpec((_ROWS, n), lambda i: (i, 0))],
        out_specs=[
            pl.BlockSpec((_ROWS, n), lambda i: (i, 0)),
            pl.BlockSpec((_ROWS, 2 * n), lambda i: (i, 0)),
        ],
        out_shape=[
            jax.ShapeDtypeStruct((b, n), jnp.float32),
            jax.ShapeDtypeStruct((b, 2 * n), jnp.float32),
        ],
        compiler_params=pltpu.CompilerParams(
            dimension_semantics=("parallel",),
        ),
    )(scores)
    rho = jnp.full((b,), 0.5, jnp.float32)
    return (w_out, rho, p_out)


# speedup vs baseline: 24.3828x; 1.6409x over previous
import functools

import jax
import jax.numpy as jnp
from jax.experimental import pallas as pl
from jax.experimental.pallas import tpu as pltpu

_K = 256
_ROWS = 8


def _sortable_i32(x):
    b = jax.lax.bitcast_convert_type(x, jnp.int32)
    return b ^ ((b >> 31) & jnp.int32(0x7FFFFFFF))


def _tc_body(s_ref, p_ref, w_ref):
    s = s_ref[...]
    rows, n = s.shape
    nchunk = n // 128

    m = jnp.max(s, axis=1, keepdims=True)
    e = jnp.exp(s - m)
    z = jnp.sum(e, axis=1, keepdims=True)
    p_ref[...] = e / z

    l = jnp.sign(s) * (1.0 - s)
    ml_ = jnp.max(l, axis=1, keepdims=True)

    kw = _sortable_i32(s)
    kl = _sortable_i32(l)

    kk = jnp.int32(_K)

    cmaxw = jnp.max(s.reshape(rows, nchunk, 128), axis=2)
    lbw = _sortable_i32(jnp.min(cmaxw, axis=1, keepdims=True))
    cmaxl = jnp.max(l.reshape(rows, nchunk, 128), axis=2)
    lbl = _sortable_i32(jnp.min(cmaxl, axis=1, keepdims=True))
    ubw = _sortable_i32(m) + 1
    ubl = _sortable_i32(ml_) + 1

    cw_lo0 = jnp.sum((kw >= lbw).astype(jnp.int32), axis=1, keepdims=True)
    cl_lo0 = jnp.sum((kl >= lbl).astype(jnp.int32), axis=1, keepdims=True)
    zero = jnp.zeros((rows, 1), jnp.int32)

    def vcond(carry):
        lw, hw, ll, hl = carry[0], carry[1], carry[2], carry[3]
        return jnp.any(hw > lw + 1) | jnp.any(hl > ll + 1)

    def vstep(carry):
        lw, hw, ll, hl, cwlo, cwhi, cllo, clhi = carry
        mw = (lw >> 1) + (hw >> 1) + (lw & hw & 1)
        ml2 = (ll >> 1) + (hl >> 1) + (ll & hl & 1)
        cw = jnp.sum((kw >= mw).astype(jnp.int32), axis=1, keepdims=True)
        cl = jnp.sum((kl >= ml2).astype(jnp.int32), axis=1, keepdims=True)
        pw = cw >= kk
        pl_ = cl >= kk
        dw = hw > lw + 1
        dl = hl > ll + 1
        lw = jnp.where(dw & pw, mw, lw)
        hw = jnp.where(dw & ~pw, mw, hw)
        cwlo = jnp.where(dw & pw, cw, cwlo)
        cwhi = jnp.where(dw & ~pw, cw, cwhi)
        ll = jnp.where(dl & pl_, ml2, ll)
        hl = jnp.where(dl & ~pl_, ml2, hl)
        cllo = jnp.where(dl & pl_, cl, cllo)
        clhi = jnp.where(dl & ~pl_, cl, clhi)
        return lw, hw, ll, hl, cwlo, cwhi, cllo, clhi

    lw, _, ll, _, cgew, cgtw, cgel, cgtl = jax.lax.while_loop(
        vcond, vstep,
        (lbw, ubw, lbl, ubl, cw_lo0, zero, cl_lo0, zero))

    rw = kk - cgtw
    rl = kk - cgtl

    idx = jax.lax.broadcasted_iota(jnp.int32, (rows, n), 1)
    eqw = kw == lw
    eql = kl == ll

    needs = jnp.any(cgew > kk) | jnp.any(cgel > kk)

    li0 = jnp.full((rows, 1), -1, jnp.int32)
    hi0i = jnp.full((rows, 1), n - 1, jnp.int32)

    def icond(carry):
        step = carry[4]
        return needs & (step < 15)

    def istep(carry):
        liw, hiw, lil, hil, step = carry
        miw = (liw + hiw) >> 1
        mil = (lil + hil) >> 1
        cw = jnp.sum((eqw & (idx <= miw)).astype(jnp.int32), axis=1,
                     keepdims=True)
        cl = jnp.sum((eql & (idx <= mil)).astype(jnp.int32), axis=1,
                     keepdims=True)
        pw = cw >= rw
        pl_ = cl >= rl
        hiw = jnp.where(pw, miw, hiw)
        liw = jnp.where(pw, liw, miw)
        hil = jnp.where(pl_, mil, hil)
        lil = jnp.where(pl_, lil, mil)
        return liw, hiw, lil, hil, step + 1

    _, itw, _, itl, _ = jax.lax.while_loop(
        icond, istep, (li0, hi0i, li0, hi0i, jnp.int32(0)))

    maskw = (kw > lw) | (eqw & (idx <= itw))
    maskl = (kl > ll) | (eql & (idx <= itl))

    ew = jnp.where(maskw, e, 0.0)
    zw = jnp.sum(ew, axis=1, keepdims=True)
    w_ref[:, 0:n] = ew / zw

    el = jnp.where(maskl, jnp.exp(l - ml_), 0.0)
    zl = jnp.sum(el, axis=1, keepdims=True)
    w_ref[:, n:2 * n] = el / zl


@jax.jit
def kernel(scores):
    b, n = scores.shape
    grid = b // _ROWS
    p_out, w_out = pl.pallas_call(
        _tc_body,
        grid=(grid,),
        in_specs=[pl.BlockSpec((_ROWS, n), lambda i: (i, 0))],
        out_specs=[
            pl.BlockSpec((_ROWS, n), lambda i: (i, 0)),
            pl.BlockSpec((_ROWS, 2 * n), lambda i: (i, 0)),
        ],
        out_shape=[
            jax.ShapeDtypeStruct((b, n), jnp.float32),
            jax.ShapeDtypeStruct((b, 2 * n), jnp.float32),
        ],
        compiler_params=pltpu.CompilerParams(
            dimension_semantics=("parallel",),
        ),
    )(scores)
    rho = jnp.full((b,), 0.5, jnp.float32)
    return (w_out, rho, p_out)


# speedup vs baseline: 28.1905x; 1.1562x over previous
import functools

import jax
import jax.numpy as jnp
from jax.experimental import pallas as pl
from jax.experimental.pallas import tpu as pltpu

_K = 256
_ROWS = 16


def _sortable_i32(x):
    b = jax.lax.bitcast_convert_type(x, jnp.int32)
    return b ^ ((b >> 31) & jnp.int32(0x7FFFFFFF))


def _cellmax(x):
    v = x
    while v.shape[1] > 256:
        h = v.shape[1] // 2
        v = jnp.maximum(v[:, :h], v[:, h:])
    return v


def _tc_body(s_ref, p_ref, w_ref):
    s = s_ref[...]
    rows, n = s.shape

    m = jnp.max(s, axis=1, keepdims=True)
    e = jnp.exp(s - m)
    z = jnp.sum(e, axis=1, keepdims=True)
    p_ref[...] = e / z

    l = jnp.sign(s) * (1.0 - s)
    ml_ = jnp.max(l, axis=1, keepdims=True)

    kw = _sortable_i32(s)
    kl = _sortable_i32(l)

    kk = jnp.int32(_K)

    lbw = _sortable_i32(jnp.min(_cellmax(s), axis=1, keepdims=True))
    lbl = _sortable_i32(jnp.min(_cellmax(l), axis=1, keepdims=True))
    ubw = _sortable_i32(m) + 1
    ubl = _sortable_i32(ml_) + 1

    cw_lo0 = jnp.sum((kw >= lbw).astype(jnp.int32), axis=1, keepdims=True)
    cl_lo0 = jnp.sum((kl >= lbl).astype(jnp.int32), axis=1, keepdims=True)
    zero = jnp.zeros((rows, 1), jnp.int32)

    def vcond(carry):
        lw, hw, ll, hl = carry[0], carry[1], carry[2], carry[3]
        return jnp.any(hw > lw + 1) | jnp.any(hl > ll + 1)

    def vstep(carry):
        lw, hw, ll, hl, cwlo, cwhi, cllo, clhi = carry
        mw = (lw >> 1) + (hw >> 1) + (lw & hw & 1)
        ml2 = (ll >> 1) + (hl >> 1) + (ll & hl & 1)
        cw = jnp.sum((kw >= mw).astype(jnp.int32), axis=1, keepdims=True)
        cl = jnp.sum((kl >= ml2).astype(jnp.int32), axis=1, keepdims=True)
        pw = cw >= kk
        pl_ = cl >= kk
        dw = hw > lw + 1
        dl = hl > ll + 1
        lw = jnp.where(dw & pw, mw, lw)
        hw = jnp.where(dw & ~pw, mw, hw)
        cwlo = jnp.where(dw & pw, cw, cwlo)
        cwhi = jnp.where(dw & ~pw, cw, cwhi)
        ll = jnp.where(dl & pl_, ml2, ll)
        hl = jnp.where(dl & ~pl_, ml2, hl)
        cllo = jnp.where(dl & pl_, cl, cllo)
        clhi = jnp.where(dl & ~pl_, cl, clhi)
        return lw, hw, ll, hl, cwlo, cwhi, cllo, clhi

    lw, _, ll, _, cgew, cgtw, cgel, cgtl = jax.lax.while_loop(
        vcond, vstep,
        (lbw, ubw, lbl, ubl, cw_lo0, zero, cl_lo0, zero))

    rw = kk - cgtw
    rl = kk - cgtl

    idx = jax.lax.broadcasted_iota(jnp.int32, (rows, n), 1)
    eqw = kw == lw
    eql = kl == ll

    needs = jnp.any(cgew > kk) | jnp.any(cgel > kk)

    li0 = jnp.full((rows, 1), -1, jnp.int32)
    hi0i = jnp.full((rows, 1), n - 1, jnp.int32)

    def icond(carry):
        step = carry[4]
        return needs & (step < 15)

    def istep(carry):
        liw, hiw, lil, hil, step = carry
        miw = (liw + hiw) >> 1
        mil = (lil + hil) >> 1
        cw = jnp.sum((eqw & (idx <= miw)).astype(jnp.int32), axis=1,
                     keepdims=True)
        cl = jnp.sum((eql & (idx <= mil)).astype(jnp.int32), axis=1,
                     keepdims=True)
        pw = cw >= rw
        pl_ = cl >= rl
        hiw = jnp.where(pw, miw, hiw)
        liw = jnp.where(pw, liw, miw)
        hil = jnp.where(pl_, mil, hil)
        lil = jnp.where(pl_, lil, mil)
        return liw, hiw, lil, hil, step + 1

    _, itw, _, itl, _ = jax.lax.while_loop(
        icond, istep, (li0, hi0i, li0, hi0i, jnp.int32(0)))

    maskw = (kw > lw) | (eqw & (idx <= itw))
    maskl = (kl > ll) | (eql & (idx <= itl))

    ew = jnp.where(maskw, e, 0.0)
    zw = jnp.sum(ew, axis=1, keepdims=True)
    w_ref[:, 0:n] = ew / zw

    el = jnp.where(maskl, jnp.exp(l - ml_), 0.0)
    zl = jnp.sum(el, axis=1, keepdims=True)
    w_ref[:, n:2 * n] = el / zl


@jax.jit
def kernel(scores):
    b, n = scores.shape
    grid = b // _ROWS
    p_out, w_out = pl.pallas_call(
        _tc_body,
        grid=(grid,),
        in_specs=[pl.BlockSpec((_ROWS, n), lambda i: (i, 0))],
        out_specs=[
            pl.BlockSpec((_ROWS, n), lambda i: (i, 0)),
            pl.BlockSpec((_ROWS, 2 * n), lambda i: (i, 0)),
        ],
        out_shape=[
            jax.ShapeDtypeStruct((b, n), jnp.float32),
            jax.ShapeDtypeStruct((b, 2 * n), jnp.float32),
        ],
        compiler_params=pltpu.CompilerParams(
            dimension_semantics=("parallel",),
        ),
    )(scores)
    rho = jnp.full((b,), 0.5, jnp.float32)
    return (w_out, rho, p_out)


# speedup vs baseline: 31.8195x; 1.1287x over previous
import functools

import jax
import jax.numpy as jnp
from jax.experimental import pallas as pl
from jax.experimental.pallas import tpu as pltpu

_K = 256
_ROWS = 32


def _sortable_i32(x):
    b = jax.lax.bitcast_convert_type(x, jnp.int32)
    return b ^ ((b >> 31) & jnp.int32(0x7FFFFFFF))


def _cellmax(x):
    v = x
    while v.shape[1] > 256:
        h = v.shape[1] // 2
        v = jnp.maximum(v[:, :h], v[:, h:])
    return v


def _tc_body(s_ref, p_ref, w_ref):
    s = s_ref[...]
    rows, n = s.shape

    m = jnp.max(s, axis=1, keepdims=True)
    e = jnp.exp(s - m)
    z = jnp.sum(e, axis=1, keepdims=True)
    p_ref[...] = e / z

    l = jnp.sign(s) * (1.0 - s)
    ml_ = jnp.max(l, axis=1, keepdims=True)

    kw = _sortable_i32(s)
    kl = _sortable_i32(l)

    kk = jnp.int32(_K)

    lbw = _sortable_i32(jnp.min(_cellmax(s), axis=1, keepdims=True))
    lbl = _sortable_i32(jnp.min(_cellmax(l), axis=1, keepdims=True))
    ubw = _sortable_i32(m) + 1
    ubl = _sortable_i32(ml_) + 1

    cw_lo0 = jnp.sum((kw >= lbw).astype(jnp.int32), axis=1, keepdims=True)
    cl_lo0 = jnp.sum((kl >= lbl).astype(jnp.int32), axis=1, keepdims=True)
    zero = jnp.zeros((rows, 1), jnp.int32)

    def vcond(carry):
        lw, hw, ll, hl = carry[0], carry[1], carry[2], carry[3]
        return jnp.any(hw > lw + 1) | jnp.any(hl > ll + 1)

    def vstep(carry):
        lw, hw, ll, hl, cwlo, cwhi, cllo, clhi = carry
        mw = (lw >> 1) + (hw >> 1) + (lw & hw & 1)
        ml2 = (ll >> 1) + (hl >> 1) + (ll & hl & 1)
        cw = jnp.sum((kw >= mw).astype(jnp.int32), axis=1, keepdims=True)
        cl = jnp.sum((kl >= ml2).astype(jnp.int32), axis=1, keepdims=True)
        pw = cw >= kk
        pl_ = cl >= kk
        dw = hw > lw + 1
        dl = hl > ll + 1
        lw = jnp.where(dw & pw, mw, lw)
        hw = jnp.where(dw & ~pw, mw, hw)
        cwlo = jnp.where(dw & pw, cw, cwlo)
        cwhi = jnp.where(dw & ~pw, cw, cwhi)
        ll = jnp.where(dl & pl_, ml2, ll)
        hl = jnp.where(dl & ~pl_, ml2, hl)
        cllo = jnp.where(dl & pl_, cl, cllo)
        clhi = jnp.where(dl & ~pl_, cl, clhi)
        return lw, hw, ll, hl, cwlo, cwhi, cllo, clhi

    lw, _, ll, _, cgew, cgtw, cgel, cgtl = jax.lax.while_loop(
        vcond, vstep,
        (lbw, ubw, lbl, ubl, cw_lo0, zero, cl_lo0, zero))

    rw = kk - cgtw
    rl = kk - cgtl

    idx = jax.lax.broadcasted_iota(jnp.int32, (rows, n), 1)
    eqw = kw == lw
    eql = kl == ll

    needs = jnp.any(cgew > kk) | jnp.any(cgel > kk)

    li0 = jnp.full((rows, 1), -1, jnp.int32)
    hi0i = jnp.full((rows, 1), n - 1, jnp.int32)

    def icond(carry):
        step = carry[4]
        return needs & (step < 15)

    def istep(carry):
        liw, hiw, lil, hil, step = carry
        miw = (liw + hiw) >> 1
        mil = (lil + hil) >> 1
        cw = jnp.sum((eqw & (idx <= miw)).astype(jnp.int32), axis=1,
                     keepdims=True)
        cl = jnp.sum((eql & (idx <= mil)).astype(jnp.int32), axis=1,
                     keepdims=True)
        pw = cw >= rw
        pl_ = cl >= rl
        hiw = jnp.where(pw, miw, hiw)
        liw = jnp.where(pw, liw, miw)
        hil = jnp.where(pl_, mil, hil)
        lil = jnp.where(pl_, lil, mil)
        return liw, hiw, lil, hil, step + 1

    _, itw, _, itl, _ = jax.lax.while_loop(
        icond, istep, (li0, hi0i, li0, hi0i, jnp.int32(0)))

    maskw = (kw > lw) | (eqw & (idx <= itw))
    maskl = (kl > ll) | (eql & (idx <= itl))

    ew = jnp.where(maskw, e, 0.0)
    zw = jnp.sum(ew, axis=1, keepdims=True)
    w_ref[:, 0:n] = ew / zw

    el = jnp.where(maskl, jnp.exp(l - ml_), 0.0)
    zl = jnp.sum(el, axis=1, keepdims=True)
    w_ref[:, n:2 * n] = el / zl


@jax.jit
def kernel(scores):
    b, n = scores.shape
    grid = b // _ROWS
    p_out, w_out = pl.pallas_call(
        _tc_body,
        grid=(grid,),
        in_specs=[pl.BlockSpec((_ROWS, n), lambda i: (i, 0))],
        out_specs=[
            pl.BlockSpec((_ROWS, n), lambda i: (i, 0)),
            pl.BlockSpec((_ROWS, 2 * n), lambda i: (i, 0)),
        ],
        out_shape=[
            jax.ShapeDtypeStruct((b, n), jnp.float32),
            jax.ShapeDtypeStruct((b, 2 * n), jnp.float32),
        ],
        compiler_params=pltpu.CompilerParams(
            dimension_semantics=("parallel",),
        ),
    )(scores)
    rho = jnp.full((b,), 0.5, jnp.float32)
    return (w_out, rho, p_out)
